# baseline (speedup 1.0000x reference)
import jax
import jax.numpy as jnp
from jax import lax
from jax.experimental import pallas as pl
from jax.experimental.pallas import tpu as pltpu

TOP_K = 8
NUM_EXPERTS = 64
HIDDEN_DIM = 4096
CHUNK = 1024
N_BUF = 3


def _route_chunk(x, w, logits_out, topv_out, topi_out):
    lt = lax.dot_general(
        w, x, (((1,), (1,)), ((), ())), preferred_element_type=jnp.float32
    )
    r = lax.broadcasted_iota(jnp.int32, (NUM_EXPERTS, NUM_EXPERTS), 0)
    c = lax.broadcasted_iota(jnp.int32, (NUM_EXPERTS, NUM_EXPERTS), 1)
    eye = (r == c).astype(jnp.float32)
    logits_out[...] = lax.dot_general(
        lt, eye, (((0,), (0,)), ((), ())), preferred_element_type=jnp.float32
    )

    eiota = lax.broadcasted_iota(jnp.int32, (NUM_EXPERTS, CHUNK), 0)
    work = lt
    vals, idxs = [], []
    for _ in range(TOP_K):
        m = jnp.max(work, axis=0, keepdims=True)
        idx = jnp.min(
            jnp.where(work == m, eiota, NUM_EXPERTS), axis=0, keepdims=True
        )
        vals.append(m)
        idxs.append(idx)
        work = jnp.where(eiota == idx, -jnp.inf, work)
    topv = jnp.concatenate(vals, axis=0)
    topi = jnp.concatenate(idxs, axis=0)

    e = jnp.exp(topv - topv[0:1, :])
    topv_out[...] = e / jnp.sum(e, axis=0, keepdims=True)
    topi_out[...] = topi


def _router(
    hs_ref,
    w_ref,
    logits_ref,
    topv_ref,
    topi_ref,
    xbuf,
    lbuf,
    vbuf,
    ibuf,
    in_sems,
    l_sems,
    v_sems,
    i_sems,
):
    n_chunks = hs_ref.shape[0] // CHUNK

    def in_copy(chunk, slot):
        return pltpu.make_async_copy(
            hs_ref.at[pl.ds(chunk * CHUNK, CHUNK), :],
            xbuf.at[slot],
            in_sems.at[slot],
        )

    def out_copies(chunk, slot):
        return (
            pltpu.make_async_copy(
                lbuf.at[slot],
                logits_ref.at[pl.ds(chunk * CHUNK, CHUNK), :],
                l_sems.at[slot],
            ),
            pltpu.make_async_copy(
                vbuf.at[slot],
                topv_ref.at[:, pl.ds(chunk * CHUNK, CHUNK)],
                v_sems.at[slot],
            ),
            pltpu.make_async_copy(
                ibuf.at[slot],
                topi_ref.at[:, pl.ds(chunk * CHUNK, CHUNK)],
                i_sems.at[slot],
            ),
        )

    for b in range(N_BUF):
        in_copy(b, b).start()

    w = w_ref[...]

    def body(chunk, carry):
        slot = lax.rem(chunk, N_BUF)
        in_copy(chunk, slot).wait()

        @pl.when(chunk >= N_BUF)
        def _():
            for cp in out_copies(chunk - N_BUF, slot):
                cp.wait()

        _route_chunk(
            xbuf[slot], w, lbuf.at[slot], vbuf.at[slot], ibuf.at[slot]
        )
        for cp in out_copies(chunk, slot):
            cp.start()

        @pl.when(chunk + N_BUF < n_chunks)
        def _():
            in_copy(chunk + N_BUF, slot).start()

        return carry

    lax.fori_loop(0, n_chunks, body, 0)

    for b in range(N_BUF):
        chunk = n_chunks - N_BUF + b
        for cp in out_copies(chunk, lax.rem(chunk, N_BUF)):
            cp.wait()


def kernel(hidden_states, weight):
    n_tokens = hidden_states.shape[0]

    logits, topv_t, topi_t = pl.pallas_call(
        _router,
        in_specs=[
            pl.BlockSpec(memory_space=pl.ANY),
            pl.BlockSpec((NUM_EXPERTS, HIDDEN_DIM), lambda: (0, 0)),
        ],
        out_specs=[
            pl.BlockSpec(memory_space=pl.ANY),
            pl.BlockSpec(memory_space=pl.ANY),
            pl.BlockSpec(memory_space=pl.ANY),
        ],
        out_shape=[
            jax.ShapeDtypeStruct((n_tokens, NUM_EXPERTS), jnp.float32),
            jax.ShapeDtypeStruct((TOP_K, n_tokens), jnp.float32),
            jax.ShapeDtypeStruct((TOP_K, n_tokens), jnp.int32),
        ],
        scratch_shapes=[
            pltpu.VMEM((N_BUF, CHUNK, HIDDEN_DIM), jnp.float32),
            pltpu.VMEM((N_BUF, CHUNK, NUM_EXPERTS), jnp.float32),
            pltpu.VMEM((N_BUF, TOP_K, CHUNK), jnp.float32),
            pltpu.VMEM((N_BUF, TOP_K, CHUNK), jnp.int32),
            pltpu.SemaphoreType.DMA((N_BUF,)),
            pltpu.SemaphoreType.DMA((N_BUF,)),
            pltpu.SemaphoreType.DMA((N_BUF,)),
            pltpu.SemaphoreType.DMA((N_BUF,)),
        ],
    )(hidden_states, weight)
    return (logits, topv_t.T, topi_t.T)
